# baseline (speedup 1.0000x reference)
import functools

import jax
import jax.numpy as jnp
from jax import lax
from jax.experimental import pallas as pl
from jax.experimental.pallas import tpu as pltpu
from jax.experimental.pallas import tpu_sc as plsc

_N = 10000
_E = 320000
_D = 128

_NC = 2
_NS = 16
_NW = _NC * _NS
_GRP = 128
_GPT = 80
_E_PAD = _NW * _GPT * _GRP
_N_PAD = 10240
_DUMMY = _N
_RPS = _N_PAD // _NS
_ZCH = _RPS // _GRP

_mesh = plsc.VectorSubcoreMesh(core_axis_name="c", subcore_axis_name="s")


@functools.partial(
    pl.kernel,
    out_type=jax.ShapeDtypeStruct((_NC, _N_PAD, _D), jnp.float32),
    mesh=_mesh,
    scratch_types=[
        pltpu.VMEM((_GPT, _GRP), jnp.int32),
        pltpu.VMEM((_GRP, _D), jnp.float32),
        pltpu.VMEM_SHARED((_N_PAD, _D), jnp.float32),
    ],
)
def _deg_kernel(dst_hbm, ones_hbm, zeros_hbm, out_hbm, idx_v, buf, acc):
    cid = lax.axis_index("c")
    sid = lax.axis_index("s")
    wid = sid * _NC + cid
    pltpu.sync_copy(zeros_hbm, buf)
    base = sid * _RPS
    for k in range(_ZCH):
        pltpu.sync_copy(buf, acc.at[pl.ds(base + k * _GRP, _GRP)])
    pltpu.sync_copy(ones_hbm, buf)
    plsc.subcore_barrier()
    pltpu.sync_copy(dst_hbm.at[wid], idx_v)

    @pl.loop(0, _GPT)
    def _(j):
        pltpu.sync_copy(buf, acc.at[idx_v.at[j]], add=True)

    plsc.subcore_barrier()
    pltpu.sync_copy(acc.at[pl.ds(base, _RPS)], out_hbm.at[cid].at[pl.ds(base, _RPS)])


@functools.partial(
    pl.kernel,
    out_type=jax.ShapeDtypeStruct((_NC, _N_PAD, _D), jnp.float32),
    mesh=_mesh,
    scratch_types=[
        pltpu.VMEM((_GPT // 2, _GRP), jnp.int32),
        pltpu.VMEM((_GPT // 2, _GRP), jnp.int32),
        pltpu.VMEM((_GRP, _D), jnp.float32),
        pltpu.VMEM((_GRP, _D), jnp.float32),
        pltpu.VMEM_SHARED((_N_PAD, _D), jnp.float32),
        pltpu.SemaphoreType.DMA,
        pltpu.SemaphoreType.DMA,
    ],
)
def _agg_kernel(g_hbm, src_hbm, dst_hbm, zeros_hbm, out_hbm,
                idx_s, idx_d, rows_a, rows_b, acc, sem_a, sem_b):
    cid = lax.axis_index("c")
    sid = lax.axis_index("s")
    wid = sid * _NC + cid
    hg = _GPT // 2
    pltpu.sync_copy(zeros_hbm, rows_a)
    base = sid * _RPS
    for k in range(_ZCH):
        pltpu.sync_copy(rows_a, acc.at[pl.ds(base + k * _GRP, _GRP)])
    plsc.subcore_barrier()

    for h in range(2):
        pltpu.sync_copy(src_hbm.at[wid].at[pl.ds(h * hg, hg)], idx_s)
        pltpu.sync_copy(dst_hbm.at[wid].at[pl.ds(h * hg, hg)], idx_d)
        @pl.loop(0, hg)
        def _(j):
            pltpu.async_copy(g_hbm.at[idx_s.at[j]], rows_a, sem_a).wait()
            pltpu.sync_copy(rows_a, acc.at[idx_d.at[j]], add=True)

    plsc.subcore_barrier()
    pltpu.sync_copy(acc.at[pl.ds(base, _RPS)], out_hbm.at[cid].at[pl.ds(base, _RPS)])


_BLK_MM = 1280


def _gemm_scale_body(x_ref, w_ref, d0_ref, d1_ref, g_ref):
    h = lax.dot_general(x_ref[...], w_ref[...], (((1,), (0,)), ((), ())),
                        preferred_element_type=jnp.float32,
                        precision=lax.Precision.HIGHEST)
    deg = 1.0 + d0_ref[:, 0:1] + d1_ref[:, 0:1]
    g_ref[...] = h * lax.rsqrt(deg)


_gemm_scale = pl.pallas_call(
    _gemm_scale_body,
    out_shape=jax.ShapeDtypeStruct((_N_PAD, _D), jnp.float32),
    grid=(_N_PAD // _BLK_MM,),
    in_specs=[
        pl.BlockSpec((_BLK_MM, _D), lambda i: (i, 0)),
        pl.BlockSpec((_D, _D), lambda i: (0, 0)),
        pl.BlockSpec((_BLK_MM, _D), lambda i: (i, 0)),
        pl.BlockSpec((_BLK_MM, _D), lambda i: (i, 0)),
    ],
    out_specs=pl.BlockSpec((_BLK_MM, _D), lambda i: (i, 0)),
)


_BLK_EP = 2000


def _epilogue_body(s0_ref, s1_ref, g_ref, d0_ref, d1_ref, b_ref, x_ref, o_ref):
    deg = 1.0 + d0_ref[:, 0:1] + d1_ref[:, 0:1]
    dinv = lax.rsqrt(deg)
    conv = dinv * (s0_ref[...] + s1_ref[...] + g_ref[...]) + b_ref[...]
    o_ref[...] = jnp.maximum(conv, 0.0) + x_ref[...]


_epilogue = pl.pallas_call(
    _epilogue_body,
    out_shape=jax.ShapeDtypeStruct((_N, _D), jnp.float32),
    grid=(_N // _BLK_EP,),
    in_specs=[
        pl.BlockSpec((_BLK_EP, _D), lambda i: (i, 0)),
        pl.BlockSpec((_BLK_EP, _D), lambda i: (i, 0)),
        pl.BlockSpec((_BLK_EP, _D), lambda i: (i, 0)),
        pl.BlockSpec((_BLK_EP, _D), lambda i: (i, 0)),
        pl.BlockSpec((_BLK_EP, _D), lambda i: (i, 0)),
        pl.BlockSpec((1, _D), lambda i: (0, 0)),
        pl.BlockSpec((_BLK_EP, _D), lambda i: (i, 0)),
    ],
    out_specs=pl.BlockSpec((_BLK_EP, _D), lambda i: (i, 0)),
)


def kernel(x, edge_index, W, b):
    src = edge_index[0].astype(jnp.int32)
    dst = edge_index[1].astype(jnp.int32)
    pad = _E_PAD - _E
    fill = jnp.full((pad,), _DUMMY, jnp.int32)
    src_g = jnp.concatenate([src, fill]).reshape(_NW, _GPT, _GRP)
    dst_g = jnp.concatenate([dst, fill]).reshape(_NW, _GPT, _GRP)
    x_pad = jnp.pad(x, ((0, _N_PAD - _N), (0, 0)))

    ones128 = jnp.ones((_GRP, _D), jnp.float32)
    zeros128 = jnp.zeros((_GRP, _D), jnp.float32)

    deg_part = _deg_kernel(dst_g, ones128, zeros128)
    d0, d1 = deg_part[0], deg_part[1]
    g = _gemm_scale(x_pad, W, d0, d1)
    s_part = _agg_kernel(g, src_g, dst_g, zeros128)
    out = _epilogue(s_part[0], s_part[1], g, d0, d1,
                    b.reshape(1, _D), x)
    return out


# speedup vs baseline: 2.8864x; 2.8864x over previous
import functools

import jax
import jax.numpy as jnp
from jax import lax
from jax.experimental import pallas as pl
from jax.experimental.pallas import tpu as pltpu
from jax.experimental.pallas import tpu_sc as plsc

_N = 10000
_E = 320000
_D = 128

_NC = 2
_NS = 16
_NW = _NC * _NS
_GRP = 128
_GPT = 80
_E_PAD = _NW * _GPT * _GRP
_N_PAD = 10240
_DUMMY = _N
_RPS = _N_PAD // _NS
_ZCH = _RPS // _GRP

_mesh = plsc.VectorSubcoreMesh(core_axis_name="c", subcore_axis_name="s")


@functools.partial(
    pl.kernel,
    out_type=jax.ShapeDtypeStruct((_NC, _N_PAD, _D), jnp.float32),
    mesh=_mesh,
    scratch_types=[
        pltpu.VMEM((_GPT, _GRP), jnp.int32),
        pltpu.VMEM((_GRP, _D), jnp.float32),
        pltpu.VMEM_SHARED((_N_PAD, _D), jnp.float32),
    ],
)
def _deg_kernel(dst_hbm, ones_hbm, zeros_hbm, out_hbm, idx_v, buf, acc):
    cid = lax.axis_index("c")
    sid = lax.axis_index("s")
    wid = sid * _NC + cid
    pltpu.sync_copy(zeros_hbm, buf)
    base = sid * _RPS
    for k in range(_ZCH):
        pltpu.sync_copy(buf, acc.at[pl.ds(base + k * _GRP, _GRP)])
    pltpu.sync_copy(ones_hbm, buf)
    plsc.subcore_barrier()
    pltpu.sync_copy(dst_hbm.at[wid], idx_v)

    @pl.loop(0, _GPT)
    def _(j):
        pltpu.sync_copy(buf, acc.at[idx_v.at[j]], add=True)

    plsc.subcore_barrier()
    pltpu.sync_copy(acc.at[pl.ds(base, _RPS)], out_hbm.at[cid].at[pl.ds(base, _RPS)])


@functools.partial(
    pl.kernel,
    out_type=jax.ShapeDtypeStruct((_NC, _N_PAD, _D), jnp.float32),
    mesh=_mesh,
    scratch_types=[
        pltpu.VMEM((_GPT // 2, _GRP), jnp.int32),
        pltpu.VMEM((_GPT // 2, _GRP), jnp.int32),
        pltpu.VMEM((_GRP, _D), jnp.float32),
        pltpu.VMEM((_GRP, _D), jnp.float32),
        pltpu.VMEM_SHARED((_N_PAD, _D), jnp.float32),
        pltpu.SemaphoreType.DMA,
        pltpu.SemaphoreType.DMA,
    ],
)
def _agg_kernel(g_hbm, src_hbm, dst_hbm, zeros_hbm, out_hbm,
                idx_s, idx_d, rows_a, rows_b, acc, sem_a, sem_b):
    cid = lax.axis_index("c")
    sid = lax.axis_index("s")
    wid = sid * _NC + cid
    hg = _GPT // 2
    pltpu.sync_copy(zeros_hbm, rows_a)
    base = sid * _RPS
    for k in range(_ZCH):
        pltpu.sync_copy(rows_a, acc.at[pl.ds(base + k * _GRP, _GRP)])
    plsc.subcore_barrier()

    for h in range(2):
        pltpu.sync_copy(src_hbm.at[wid].at[pl.ds(h * hg, hg)], idx_s)
        pltpu.sync_copy(dst_hbm.at[wid].at[pl.ds(h * hg, hg)], idx_d)
        pltpu.make_async_copy(g_hbm.at[idx_s.at[0]], rows_a, sem_a).start()

        @pl.loop(0, hg, step=2)
        def _(j):
            pltpu.make_async_copy(g_hbm.at[idx_s.at[j + 1]], rows_b, sem_b).start()
            pltpu.make_async_copy(g_hbm.at[idx_s.at[j]], rows_a, sem_a).wait()
            pltpu.sync_copy(rows_a, acc.at[idx_d.at[j]], add=True)

            @pl.when(j + 2 < hg)
            def _():
                pltpu.make_async_copy(g_hbm.at[idx_s.at[j + 2]], rows_a, sem_a).start()

            pltpu.make_async_copy(g_hbm.at[idx_s.at[j + 1]], rows_b, sem_b).wait()
            pltpu.sync_copy(rows_b, acc.at[idx_d.at[j + 1]], add=True)

    plsc.subcore_barrier()
    pltpu.sync_copy(acc.at[pl.ds(base, _RPS)], out_hbm.at[cid].at[pl.ds(base, _RPS)])


_BLK_MM = 1280


def _gemm_scale_body(x_ref, w_ref, d0_ref, d1_ref, g_ref):
    h = lax.dot_general(x_ref[...], w_ref[...], (((1,), (0,)), ((), ())),
                        preferred_element_type=jnp.float32,
                        precision=lax.Precision.HIGHEST)
    deg = 1.0 + d0_ref[:, 0:1] + d1_ref[:, 0:1]
    g_ref[...] = h * lax.rsqrt(deg)


_gemm_scale = pl.pallas_call(
    _gemm_scale_body,
    out_shape=jax.ShapeDtypeStruct((_N_PAD, _D), jnp.float32),
    grid=(_N_PAD // _BLK_MM,),
    in_specs=[
        pl.BlockSpec((_BLK_MM, _D), lambda i: (i, 0)),
        pl.BlockSpec((_D, _D), lambda i: (0, 0)),
        pl.BlockSpec((_BLK_MM, _D), lambda i: (i, 0)),
        pl.BlockSpec((_BLK_MM, _D), lambda i: (i, 0)),
    ],
    out_specs=pl.BlockSpec((_BLK_MM, _D), lambda i: (i, 0)),
)


_BLK_EP = 2000


def _epilogue_body(s0_ref, s1_ref, g_ref, d0_ref, d1_ref, b_ref, x_ref, o_ref):
    deg = 1.0 + d0_ref[:, 0:1] + d1_ref[:, 0:1]
    dinv = lax.rsqrt(deg)
    conv = dinv * (s0_ref[...] + s1_ref[...] + g_ref[...]) + b_ref[...]
    o_ref[...] = jnp.maximum(conv, 0.0) + x_ref[...]


_epilogue = pl.pallas_call(
    _epilogue_body,
    out_shape=jax.ShapeDtypeStruct((_N, _D), jnp.float32),
    grid=(_N // _BLK_EP,),
    in_specs=[
        pl.BlockSpec((_BLK_EP, _D), lambda i: (i, 0)),
        pl.BlockSpec((_BLK_EP, _D), lambda i: (i, 0)),
        pl.BlockSpec((_BLK_EP, _D), lambda i: (i, 0)),
        pl.BlockSpec((_BLK_EP, _D), lambda i: (i, 0)),
        pl.BlockSpec((_BLK_EP, _D), lambda i: (i, 0)),
        pl.BlockSpec((1, _D), lambda i: (0, 0)),
        pl.BlockSpec((_BLK_EP, _D), lambda i: (i, 0)),
    ],
    out_specs=pl.BlockSpec((_BLK_EP, _D), lambda i: (i, 0)),
)


def kernel(x, edge_index, W, b):
    src = edge_index[0].astype(jnp.int32)
    dst = edge_index[1].astype(jnp.int32)
    pad = _E_PAD - _E
    fill = _DUMMY + (jnp.arange(pad, dtype=jnp.int32) % (_N_PAD - _N))
    src_g = jnp.concatenate([src, fill]).reshape(_NW, _GPT, _GRP)
    dst_g = jnp.concatenate([dst, fill]).reshape(_NW, _GPT, _GRP)
    x_pad = jnp.pad(x, ((0, _N_PAD - _N), (0, 0)))

    ones128 = jnp.ones((_GRP, _D), jnp.float32)
    zeros128 = jnp.zeros((_GRP, _D), jnp.float32)

    deg_part = _deg_kernel(dst_g, ones128, zeros128)
    d0, d1 = deg_part[0], deg_part[1]
    g = _gemm_scale(x_pad, W, d0, d1)
    s_part = _agg_kernel(g, src_g, dst_g, zeros128)
    out = _epilogue(s_part[0], s_part[1], g, d0, d1,
                    b.reshape(1, _D), x)
    return out
